# Initial kernel scaffold; baseline (speedup 1.0000x reference)
import jax
import jax.numpy as jnp
from jax.experimental import pallas as pl


def kernel(x, edge_index, W0, attn_l0, attn_r0, bias0, W1, attn_l1, attn_r1, bias1, W_res1):
    raise NotImplementedError("write your pallas kernel here")


# speedup vs baseline: 7.1447x; 7.1447x over previous
import functools

import jax
import jax.numpy as jnp
from jax import lax
from jax.experimental import pallas as pl
from jax.experimental.pallas import tpu as pltpu
from jax.experimental.pallas import tpu_sc as plsc

N = 10000
E = 320000
D = 256
H0 = 4
NSC = 2
NTILE = 16
NW = NSC * NTILE
NR = 320
NP = NW * NR
EPT = E // NW
CHK = 16000
CAP = 2048
NSLAB = E // CAP + 2
ESLOTS = NSLAB * CAP
G = 64

_sc_mesh = plsc.VectorSubcoreMesh(core_axis_name="c", subcore_axis_name="s")
_sc_params = pltpu.CompilerParams(needs_layout_passes=False)


def _route_body(src_hbm, dst_hbm, ebs_hbm, ebd_hbm, cnt_hbm,
                src_v, dst_v, stgs_v, stgd_v, sem):
    c = lax.axis_index("c")
    s = lax.axis_index("s")
    tid = c * NTILE + s
    lo = tid * NR
    lanes = lax.iota(jnp.int32, 16)

    def chunk(ci, carry):
        off, nslab = carry
        pltpu.sync_copy(src_hbm.at[pl.ds(ci * CHK, CHK)], src_v)
        pltpu.sync_copy(dst_hbm.at[pl.ds(ci * CHK, CHK)], dst_v)

        def step(i, carry):
            off, nslab = carry
            sv = src_v[pl.ds(i * 16, 16)]
            dv = dst_v[pl.ds(i * 16, 16)]
            dl = dv - lo
            m = (dl >= 0) & (dl < NR)
            scan = plsc.cumsum(m.astype(jnp.int32))
            pos = off + scan - 1
            plsc.store_scatter(stgs_v, [pos], sv, mask=m)
            plsc.store_scatter(stgd_v, [pos], dl, mask=m)
            off = off + scan[15]
            full = off >= CAP

            @pl.when(full)
            def _():
                pltpu.sync_copy(stgs_v.at[pl.ds(0, CAP)],
                                ebs_hbm.at[tid, pl.ds(nslab * CAP, CAP)])
                pltpu.sync_copy(stgd_v.at[pl.ds(0, CAP)],
                                ebd_hbm.at[tid, pl.ds(nslab * CAP, CAP)])
                tail_s = stgs_v[pl.ds(CAP, 16)]
                tail_d = stgd_v[pl.ds(CAP, 16)]
                stgs_v[pl.ds(0, 16)] = tail_s
                stgd_v[pl.ds(0, 16)] = tail_d

            off = jnp.where(full, off - CAP, off)
            nslab = jnp.where(full, nslab + 1, nslab)
            return off, nslab

        return lax.fori_loop(0, CHK // 16, step, (off, nslab))

    off, nslab = lax.fori_loop(0, E // CHK, chunk,
                               (jnp.int32(0), jnp.int32(0)))
    pltpu.sync_copy(stgs_v.at[pl.ds(0, CAP)],
                    ebs_hbm.at[tid, pl.ds(nslab * CAP, CAP)])
    pltpu.sync_copy(stgd_v.at[pl.ds(0, CAP)],
                    ebd_hbm.at[tid, pl.ds(nslab * CAP, CAP)])
    cnt = nslab * CAP + off
    for k in range(8):
        stgs_v[pl.ds(k * 16, 16)] = jnp.full((16,), cnt, jnp.int32)
    pltpu.sync_copy(stgs_v.at[pl.ds(0, 128)], cnt_hbm.at[tid])


_route = functools.partial(
    pl.kernel,
    out_type=(
        jax.ShapeDtypeStruct((NW, ESLOTS), jnp.int32),
        jax.ShapeDtypeStruct((NW, ESLOTS), jnp.int32),
        jax.ShapeDtypeStruct((NW, 128), jnp.int32),
    ),
    mesh=_sc_mesh,
    compiler_params=_sc_params,
    scratch_types=[
        pltpu.VMEM((CHK,), jnp.int32),
        pltpu.VMEM((CHK,), jnp.int32),
        pltpu.VMEM((CAP + 16,), jnp.int32),
        pltpu.VMEM((CAP + 16,), jnp.int32),
        pltpu.SemaphoreType.DMA,
    ],
)(_route_body)


def _agg_body(ft_hbm, el_hbm, erp_hbm, ebs_hbm, ebd_hbm, cnt_hbm,
              rst_hbm, den_hbm,
              el_v, er_v, es_v, ed_v, gidx_v, gdst_v, rows_v, w_v,
              acc_v, den_v, cnt_v, sem):
    c = lax.axis_index("c")
    s = lax.axis_index("s")
    tid = c * NTILE + s
    lanes = lax.iota(jnp.int32, 16)

    pltpu.sync_copy(cnt_hbm.at[tid], cnt_v)
    cnt = cnt_v[pl.ds(0, 16)][0]
    pltpu.sync_copy(el_hbm, el_v)
    pltpu.sync_copy(erp_hbm.at[pl.ds(tid * NR, NR)], er_v)

    zeros16 = jnp.zeros((16,), jnp.float32)

    def zacc(j, _):
        for k in range(D // 16):
            acc_v[j, pl.ds(k * 16, 16)] = zeros16
        return 0
    lax.fori_loop(0, NR, zacc, 0)
    for k in range(NR // 16):
        den_v[pl.ds(k * 16, 16)] = zeros16

    nslabs = (cnt + (CAP - 1)) // CAP

    def slab(sl, _):
        pltpu.sync_copy(ebs_hbm.at[tid, pl.ds(sl * CAP, CAP)], es_v)
        pltpu.sync_copy(ebd_hbm.at[tid, pl.ds(sl * CAP, CAP)], ed_v)
        rem = cnt - sl * CAP
        ng = jnp.minimum((rem + (G - 1)) // G, CAP // G)

        def group(g, _):
            base = g * G
            for j4 in range(G // 16):
                sv = es_v[pl.ds(base + j4 * 16, 16)]
                dv = ed_v[pl.ds(base + j4 * 16, 16)]
                sv = jnp.minimum(jnp.maximum(sv, 0), N - 1)
                dv = jnp.minimum(jnp.maximum(dv, 0), NR - 1)
                score = (plsc.load_gather(el_v, [sv])
                         + plsc.load_gather(er_v, [dv]))
                score = jnp.where(score >= 0.0, score, 0.2 * score)
                w = jnp.exp(score)
                pos = sl * CAP + base + j4 * 16 + lanes
                w = jnp.where(pos < cnt, w, 0.0)
                w_v[pl.ds(j4 * 16, 16)] = w
                gidx_v[pl.ds(j4 * 16, 16)] = sv
                gdst_v[pl.ds(j4 * 16, 16)] = dv
                plsc.addupdate_scatter(den_v, [dv], w)
            pltpu.async_copy(ft_hbm.at[gidx_v], rows_v, sem).wait()

            def accum(j, _):
                wj = plsc.load_gather(w_v, [jnp.full((16,), j, jnp.int32)])
                dlj = plsc.load_gather(gdst_v,
                                       [jnp.full((16,), j, jnp.int32)])[0]
                for k in range(D // 16):
                    acc_v[dlj, pl.ds(k * 16, 16)] = (
                        acc_v[dlj, pl.ds(k * 16, 16)]
                        + wj * rows_v[j, pl.ds(k * 16, 16)])
                return 0
            lax.fori_loop(0, G, accum, 0)
            return 0
        lax.fori_loop(0, ng, group, 0)
        return 0
    lax.fori_loop(0, nslabs, slab, 0)

    pltpu.sync_copy(acc_v, rst_hbm.at[pl.ds(tid * NR, NR)])
    pltpu.sync_copy(den_v, den_hbm.at[tid])


_agg = functools.partial(
    pl.kernel,
    out_type=(
        jax.ShapeDtypeStruct((NP, D), jnp.float32),
        jax.ShapeDtypeStruct((NW, NR), jnp.float32),
    ),
    mesh=_sc_mesh,
    compiler_params=_sc_params,
    scratch_types=[
        pltpu.VMEM((N,), jnp.float32),
        pltpu.VMEM((NR,), jnp.float32),
        pltpu.VMEM((CAP,), jnp.int32),
        pltpu.VMEM((CAP,), jnp.int32),
        pltpu.VMEM((G,), jnp.int32),
        pltpu.VMEM((G,), jnp.int32),
        pltpu.VMEM((G, D), jnp.float32),
        pltpu.VMEM((G,), jnp.float32),
        pltpu.VMEM((NR, D), jnp.float32),
        pltpu.VMEM((NR,), jnp.float32),
        pltpu.VMEM((128,), jnp.int32),
        pltpu.SemaphoreType.DMA,
    ],
)(_agg_body)


BN = 1000


def _tc1_body(x_ref, w_ref, al_ref, ar_ref,
              f0_ref, f1_ref, f2_ref, f3_ref, el_ref, er_ref):
    ft = jnp.dot(x_ref[...], w_ref[...], preferred_element_type=jnp.float32)
    outs = (f0_ref, f1_ref, f2_ref, f3_ref)
    els, ers = [], []
    for h in range(H0):
        fth = ft[:, h * D:(h + 1) * D]
        outs[h][...] = fth
        els.append(jnp.sum(fth * al_ref[h][None, :], axis=1))
        ers.append(jnp.sum(fth * ar_ref[h][None, :], axis=1))
    el_ref[...] = jnp.stack(els, axis=1)
    er_ref[...] = jnp.stack(ers, axis=1)


def _tc1(x, W0, al0, ar0):
    fspec = pl.BlockSpec((BN, D), lambda i: (i, 0))
    espec = pl.BlockSpec((BN, H0), lambda i: (i, 0))
    return pl.pallas_call(
        _tc1_body,
        grid=(N // BN,),
        in_specs=[
            pl.BlockSpec((BN, 128), lambda i: (i, 0)),
            pl.BlockSpec((128, H0 * D), lambda i: (0, 0)),
            pl.BlockSpec((H0, D), lambda i: (0, 0)),
            pl.BlockSpec((H0, D), lambda i: (0, 0)),
        ],
        out_specs=[fspec, fspec, fspec, fspec, espec, espec],
        out_shape=[jax.ShapeDtypeStruct((N, D), jnp.float32)] * H0
        + [jax.ShapeDtypeStruct((N, H0), jnp.float32)] * 2,
    )(x, W0, al0, ar0)


def _elu(v):
    return jnp.where(v > 0.0, v, jnp.exp(jnp.minimum(v, 0.0)) - 1.0)


def _tc2_body(r0_ref, r1_ref, r2_ref, r3_ref,
              d0_ref, d1_ref, d2_ref, d3_ref,
              b0_ref, w1_ref, wr_ref, al_ref, ar_ref,
              ft_ref, res_ref, elr_ref):
    rrefs = (r0_ref, r1_ref, r2_ref, r3_ref)
    drefs = (d0_ref, d1_ref, d2_ref, d3_ref)
    hs = []
    for h in range(H0):
        den = drefs[h][...]
        den = jnp.where(den == 0.0, 1.0, den)
        hf = rrefs[h][...] / den + b0_ref[0, h * D:(h + 1) * D][None, :]
        hs.append(_elu(hf))
    hcat = jnp.concatenate(hs, axis=1)
    ft1 = jnp.dot(hcat, w1_ref[...], preferred_element_type=jnp.float32)
    res = jnp.dot(hcat, wr_ref[...], preferred_element_type=jnp.float32)
    ft_ref[...] = ft1
    res_ref[...] = res
    el1 = jnp.sum(ft1 * al_ref[0][None, :], axis=1)
    er1 = jnp.sum(ft1 * ar_ref[0][None, :], axis=1)
    elr_ref[...] = jnp.concatenate(
        [el1[:, None], er1[:, None], jnp.zeros((BN, 6), jnp.float32)], axis=1)


def _tc2(rs, ds, b0, W1, Wres, al1, ar1):
    rspec = pl.BlockSpec((BN, D), lambda i: (i, 0))
    dspec = pl.BlockSpec((BN, 1), lambda i: (i, 0))
    return pl.pallas_call(
        _tc2_body,
        grid=(N // BN,),
        in_specs=[
            rspec, rspec, rspec, rspec,
            dspec, dspec, dspec, dspec,
            pl.BlockSpec((1, H0 * D), lambda i: (0, 0)),
            pl.BlockSpec((H0 * D, D), lambda i: (0, 0)),
            pl.BlockSpec((H0 * D, D), lambda i: (0, 0)),
            pl.BlockSpec((1, D), lambda i: (0, 0)),
            pl.BlockSpec((1, D), lambda i: (0, 0)),
        ],
        out_specs=[
            pl.BlockSpec((BN, D), lambda i: (i, 0)),
            pl.BlockSpec((BN, D), lambda i: (i, 0)),
            pl.BlockSpec((BN, 8), lambda i: (i, 0)),
        ],
        out_shape=[
            jax.ShapeDtypeStruct((N, D), jnp.float32),
            jax.ShapeDtypeStruct((N, D), jnp.float32),
            jax.ShapeDtypeStruct((N, 8), jnp.float32),
        ],
    )(*rs, *ds, b0, W1, Wres, al1, ar1)


def _tc3_body(r_ref, d_ref, res_ref, b1_ref, out_ref):
    den = d_ref[...]
    den = jnp.where(den == 0.0, 1.0, den)
    o = r_ref[...] / den + res_ref[...] + b1_ref[0][None, :]
    out_ref[...] = _elu(o)


def _tc3(r, d, res, b1):
    return pl.pallas_call(
        _tc3_body,
        grid=(N // BN,),
        in_specs=[
            pl.BlockSpec((BN, D), lambda i: (i, 0)),
            pl.BlockSpec((BN, 1), lambda i: (i, 0)),
            pl.BlockSpec((BN, D), lambda i: (i, 0)),
            pl.BlockSpec((1, D), lambda i: (0, 0)),
        ],
        out_specs=pl.BlockSpec((BN, D), lambda i: (i, 0)),
        out_shape=jax.ShapeDtypeStruct((N, D), jnp.float32),
    )(r, d, res, b1)


def kernel(x, edge_index, W0, attn_l0, attn_r0, bias0,
           W1, attn_l1, attn_r1, bias1, W_res1):
    src = edge_index[0]
    dst = edge_index[1]
    al0 = attn_l0.reshape(H0, D)
    ar0 = attn_r0.reshape(H0, D)
    al1 = attn_l1.reshape(1, D)
    ar1 = attn_r1.reshape(1, D)

    ebs, ebd, cnts = _route(src, dst)

    f0, f1, f2, f3, el0, er0 = _tc1(x, W0, al0, ar0)

    rs, ds = [], []
    for h, fe in enumerate((f0, f1, f2, f3)):
        elh = el0[:, h]
        erh = jnp.pad(er0[:, h], (0, NP - N))
        rst, den = _agg(fe, elh, erh, ebs, ebd, cnts)
        rs.append(rst[:N])
        ds.append(den.reshape(NP)[:N][:, None])

    fte1, res1, elr1 = _tc2(rs, ds, bias0.reshape(1, H0 * D),
                            W1, W_res1, al1, ar1)

    el1 = elr1[:, 0]
    er1 = jnp.pad(elr1[:, 1], (0, NP - N))
    rst1, den1 = _agg(fte1, el1, er1, ebs, ebd, cnts)

    return _tc3(rst1[:N], den1.reshape(NP)[:N][:, None],
                res1, bias1.reshape(1, D))


# speedup vs baseline: 10.3188x; 1.4443x over previous
import functools

import jax
import jax.numpy as jnp
from jax import lax
from jax.experimental import pallas as pl
from jax.experimental.pallas import tpu as pltpu
from jax.experimental.pallas import tpu_sc as plsc

N = 10000
E = 320000
D = 256
H0 = 4
NSC = 2
NTILE = 16
NW = NSC * NTILE
NR = 320
NP = NW * NR
EPT = E // NW
CHK = 16000
CAP = 1920
NSLAB = E // CAP + 2
ESLOTS = NSLAB * CAP
G = 48

_sc_mesh = plsc.VectorSubcoreMesh(core_axis_name="c", subcore_axis_name="s")
_sc_params = pltpu.CompilerParams(needs_layout_passes=False)


def _route_body(src_hbm, dst_hbm, ebs_hbm, ebd_hbm, cnt_hbm,
                src_v, dst_v, stgs_v, stgd_v, sem):
    c = lax.axis_index("c")
    s = lax.axis_index("s")
    tid = c * NTILE + s
    lo = tid * NR
    lanes = lax.iota(jnp.int32, 16)

    def chunk(ci, carry):
        off, nslab = carry
        pltpu.sync_copy(src_hbm.at[pl.ds(ci * CHK, CHK)], src_v)
        pltpu.sync_copy(dst_hbm.at[pl.ds(ci * CHK, CHK)], dst_v)

        def step(i, carry):
            off, nslab = carry
            sv = src_v[pl.ds(i * 16, 16)]
            dv = dst_v[pl.ds(i * 16, 16)]
            dl = dv - lo
            m = (dl >= 0) & (dl < NR)
            scan = plsc.cumsum(m.astype(jnp.int32))
            pos = off + scan - 1
            plsc.store_scatter(stgs_v, [pos], sv, mask=m)
            plsc.store_scatter(stgd_v, [pos], dl, mask=m)
            off = off + scan[15]
            full = off >= CAP

            @pl.when(full)
            def _():
                pltpu.sync_copy(stgs_v.at[pl.ds(0, CAP)],
                                ebs_hbm.at[tid, pl.ds(nslab * CAP, CAP)])
                pltpu.sync_copy(stgd_v.at[pl.ds(0, CAP)],
                                ebd_hbm.at[tid, pl.ds(nslab * CAP, CAP)])
                tail_s = stgs_v[pl.ds(CAP, 16)]
                tail_d = stgd_v[pl.ds(CAP, 16)]
                stgs_v[pl.ds(0, 16)] = tail_s
                stgd_v[pl.ds(0, 16)] = tail_d

            off = jnp.where(full, off - CAP, off)
            nslab = jnp.where(full, nslab + 1, nslab)
            return off, nslab

        return lax.fori_loop(0, CHK // 16, step, (off, nslab))

    off, nslab = lax.fori_loop(0, E // CHK, chunk,
                               (jnp.int32(0), jnp.int32(0)))
    pltpu.sync_copy(stgs_v.at[pl.ds(0, CAP)],
                    ebs_hbm.at[tid, pl.ds(nslab * CAP, CAP)])
    pltpu.sync_copy(stgd_v.at[pl.ds(0, CAP)],
                    ebd_hbm.at[tid, pl.ds(nslab * CAP, CAP)])
    cnt = nslab * CAP + off
    for k in range(8):
        stgs_v[pl.ds(k * 16, 16)] = jnp.full((16,), cnt, jnp.int32)
    pltpu.sync_copy(stgs_v.at[pl.ds(0, 128)], cnt_hbm.at[tid])


_route = functools.partial(
    pl.kernel,
    out_type=(
        jax.ShapeDtypeStruct((NW, ESLOTS), jnp.int32),
        jax.ShapeDtypeStruct((NW, ESLOTS), jnp.int32),
        jax.ShapeDtypeStruct((NW, 128), jnp.int32),
    ),
    mesh=_sc_mesh,
    compiler_params=_sc_params,
    scratch_types=[
        pltpu.VMEM((CHK,), jnp.int32),
        pltpu.VMEM((CHK,), jnp.int32),
        pltpu.VMEM((CAP + 16,), jnp.int32),
        pltpu.VMEM((CAP + 16,), jnp.int32),
        pltpu.SemaphoreType.DMA,
    ],
)(_route_body)


def _agg_body(ft_hbm, el_hbm, erp_hbm, ebs_hbm, ebd_hbm, cnt_hbm,
              rst_hbm, den_hbm,
              el_v, er_v, es_v, ed_v, ws_v, rows0_v, rows1_v,
              acc_v, den_v, cnt_v, sem0, sem1):
    c = lax.axis_index("c")
    s = lax.axis_index("s")
    tid = c * NTILE + s
    lanes = lax.iota(jnp.int32, 16)

    pltpu.sync_copy(cnt_hbm.at[tid], cnt_v)
    cnt = cnt_v[pl.ds(0, 16)][0]
    pltpu.sync_copy(el_hbm, el_v)
    pltpu.sync_copy(erp_hbm.at[pl.ds(tid * NR, NR)], er_v)

    zeros16 = jnp.zeros((16,), jnp.float32)

    def zacc(j, _):
        for k in range(D // 16):
            acc_v[j, pl.ds(k * 16, 16)] = zeros16
        return 0
    lax.fori_loop(0, NR, zacc, 0)
    for k in range(NR // 16):
        den_v[pl.ds(k * 16, 16)] = zeros16

    nslabs = (cnt + (CAP - 1)) // CAP

    def accum(rows_ref, base):
        def j16_step(j16, _):
            wvec = ws_v[pl.ds(base + j16 * 16, 16)]
            dvec = ed_v[pl.ds(base + j16 * 16, 16)]
            for jj in range(16):
                wsp = jnp.full((16,), wvec[jj], jnp.float32)
                rowi = jnp.full((16,), dvec[jj], jnp.int32)
                j = j16 * 16 + jj
                for k in range(D // 16):
                    contrib = wsp * rows_ref[j, pl.ds(k * 16, 16)]
                    plsc.addupdate_scatter(acc_v, [rowi, lanes + k * 16],
                                           contrib)
            return 0
        lax.fori_loop(0, G // 16, j16_step, 0)

    def slab(sl, _):
        pltpu.sync_copy(ebs_hbm.at[tid, pl.ds(sl * CAP, CAP)], es_v)
        pltpu.sync_copy(ebd_hbm.at[tid, pl.ds(sl * CAP, CAP)], ed_v)

        def wstep(i, _):
            sv = es_v[pl.ds(i * 16, 16)]
            dv = ed_v[pl.ds(i * 16, 16)]
            sv = jnp.minimum(jnp.maximum(sv, 0), N - 1)
            dv = jnp.minimum(jnp.maximum(dv, 0), NR - 1)
            score = (plsc.load_gather(el_v, [sv])
                     + plsc.load_gather(er_v, [dv]))
            score = jnp.where(score >= 0.0, score, 0.2 * score)
            w = jnp.exp(score)
            pos = sl * CAP + i * 16 + lanes
            w = jnp.where(pos < cnt, w, 0.0)
            ws_v[pl.ds(i * 16, 16)] = w
            es_v[pl.ds(i * 16, 16)] = sv
            ed_v[pl.ds(i * 16, 16)] = dv
            plsc.addupdate_scatter(den_v, [dv], w)
            return 0
        lax.fori_loop(0, CAP // 16, wstep, 0)

        rem = cnt - sl * CAP
        ng = jnp.minimum((rem + (G - 1)) // G, CAP // G)

        pltpu.async_copy(ft_hbm.at[es_v.at[pl.ds(0, G)]], rows0_v, sem0)

        def group(g, _):
            nxt = g + 1
            even = (g % 2) == 0

            @pl.when(even)
            def _():
                @pl.when(nxt < ng)
                def _():
                    pltpu.async_copy(ft_hbm.at[es_v.at[pl.ds(nxt * G, G)]],
                                     rows1_v, sem1)
                pltpu.make_async_copy(ft_hbm.at[pl.ds(0, G)],
                                      rows0_v, sem0).wait()
                accum(rows0_v, g * G)

            @pl.when(jnp.logical_not(even))
            def _():
                @pl.when(nxt < ng)
                def _():
                    pltpu.async_copy(ft_hbm.at[es_v.at[pl.ds(nxt * G, G)]],
                                     rows0_v, sem0)
                pltpu.make_async_copy(ft_hbm.at[pl.ds(0, G)],
                                      rows1_v, sem1).wait()
                accum(rows1_v, g * G)
            return 0
        lax.fori_loop(0, ng, group, 0)
        return 0
    lax.fori_loop(0, nslabs, slab, 0)

    pltpu.sync_copy(acc_v, rst_hbm.at[pl.ds(tid * NR, NR)])
    pltpu.sync_copy(den_v, den_hbm.at[tid])


_agg = functools.partial(
    pl.kernel,
    out_type=(
        jax.ShapeDtypeStruct((NP, D), jnp.float32),
        jax.ShapeDtypeStruct((NW, NR), jnp.float32),
    ),
    mesh=_sc_mesh,
    compiler_params=_sc_params,
    scratch_types=[
        pltpu.VMEM((N,), jnp.float32),
        pltpu.VMEM((NR,), jnp.float32),
        pltpu.VMEM((CAP,), jnp.int32),
        pltpu.VMEM((CAP,), jnp.int32),
        pltpu.VMEM((CAP,), jnp.float32),
        pltpu.VMEM((G, D), jnp.float32),
        pltpu.VMEM((G, D), jnp.float32),
        pltpu.VMEM((NR, D), jnp.float32),
        pltpu.VMEM((NR,), jnp.float32),
        pltpu.VMEM((128,), jnp.int32),
        pltpu.SemaphoreType.DMA,
        pltpu.SemaphoreType.DMA,
    ],
)(_agg_body)


BN = 1000


def _tc1_body(x_ref, w_ref, al_ref, ar_ref,
              f0_ref, f1_ref, f2_ref, f3_ref, el_ref, er_ref):
    ft = jnp.dot(x_ref[...], w_ref[...], preferred_element_type=jnp.float32)
    outs = (f0_ref, f1_ref, f2_ref, f3_ref)
    els, ers = [], []
    for h in range(H0):
        fth = ft[:, h * D:(h + 1) * D]
        outs[h][...] = fth
        els.append(jnp.sum(fth * al_ref[h][None, :], axis=1))
        ers.append(jnp.sum(fth * ar_ref[h][None, :], axis=1))
    el_ref[...] = jnp.stack(els, axis=1)
    er_ref[...] = jnp.stack(ers, axis=1)


def _tc1(x, W0, al0, ar0):
    fspec = pl.BlockSpec((BN, D), lambda i: (i, 0))
    espec = pl.BlockSpec((BN, H0), lambda i: (i, 0))
    return pl.pallas_call(
        _tc1_body,
        grid=(N // BN,),
        in_specs=[
            pl.BlockSpec((BN, 128), lambda i: (i, 0)),
            pl.BlockSpec((128, H0 * D), lambda i: (0, 0)),
            pl.BlockSpec((H0, D), lambda i: (0, 0)),
            pl.BlockSpec((H0, D), lambda i: (0, 0)),
        ],
        out_specs=[fspec, fspec, fspec, fspec, espec, espec],
        out_shape=[jax.ShapeDtypeStruct((N, D), jnp.float32)] * H0
        + [jax.ShapeDtypeStruct((N, H0), jnp.float32)] * 2,
    )(x, W0, al0, ar0)


def _elu(v):
    return jnp.where(v > 0.0, v, jnp.exp(jnp.minimum(v, 0.0)) - 1.0)


def _tc2_body(r0_ref, r1_ref, r2_ref, r3_ref,
              d0_ref, d1_ref, d2_ref, d3_ref,
              b0_ref, w1_ref, wr_ref, al_ref, ar_ref,
              ft_ref, res_ref, elr_ref):
    rrefs = (r0_ref, r1_ref, r2_ref, r3_ref)
    drefs = (d0_ref, d1_ref, d2_ref, d3_ref)
    hs = []
    for h in range(H0):
        den = drefs[h][...]
        den = jnp.where(den == 0.0, 1.0, den)
        hf = rrefs[h][...] / den + b0_ref[0, h * D:(h + 1) * D][None, :]
        hs.append(_elu(hf))
    hcat = jnp.concatenate(hs, axis=1)
    ft1 = jnp.dot(hcat, w1_ref[...], preferred_element_type=jnp.float32)
    res = jnp.dot(hcat, wr_ref[...], preferred_element_type=jnp.float32)
    ft_ref[...] = ft1
    res_ref[...] = res
    el1 = jnp.sum(ft1 * al_ref[0][None, :], axis=1)
    er1 = jnp.sum(ft1 * ar_ref[0][None, :], axis=1)
    elr_ref[...] = jnp.concatenate(
        [el1[:, None], er1[:, None], jnp.zeros((BN, 6), jnp.float32)], axis=1)


def _tc2(rs, ds, b0, W1, Wres, al1, ar1):
    rspec = pl.BlockSpec((BN, D), lambda i: (i, 0))
    dspec = pl.BlockSpec((BN, 1), lambda i: (i, 0))
    return pl.pallas_call(
        _tc2_body,
        grid=(N // BN,),
        in_specs=[
            rspec, rspec, rspec, rspec,
            dspec, dspec, dspec, dspec,
            pl.BlockSpec((1, H0 * D), lambda i: (0, 0)),
            pl.BlockSpec((H0 * D, D), lambda i: (0, 0)),
            pl.BlockSpec((H0 * D, D), lambda i: (0, 0)),
            pl.BlockSpec((1, D), lambda i: (0, 0)),
            pl.BlockSpec((1, D), lambda i: (0, 0)),
        ],
        out_specs=[
            pl.BlockSpec((BN, D), lambda i: (i, 0)),
            pl.BlockSpec((BN, D), lambda i: (i, 0)),
            pl.BlockSpec((BN, 8), lambda i: (i, 0)),
        ],
        out_shape=[
            jax.ShapeDtypeStruct((N, D), jnp.float32),
            jax.ShapeDtypeStruct((N, D), jnp.float32),
            jax.ShapeDtypeStruct((N, 8), jnp.float32),
        ],
    )(*rs, *ds, b0, W1, Wres, al1, ar1)


def _tc3_body(r_ref, d_ref, res_ref, b1_ref, out_ref):
    den = d_ref[...]
    den = jnp.where(den == 0.0, 1.0, den)
    o = r_ref[...] / den + res_ref[...] + b1_ref[0][None, :]
    out_ref[...] = _elu(o)


def _tc3(r, d, res, b1):
    return pl.pallas_call(
        _tc3_body,
        grid=(N // BN,),
        in_specs=[
            pl.BlockSpec((BN, D), lambda i: (i, 0)),
            pl.BlockSpec((BN, 1), lambda i: (i, 0)),
            pl.BlockSpec((BN, D), lambda i: (i, 0)),
            pl.BlockSpec((1, D), lambda i: (0, 0)),
        ],
        out_specs=pl.BlockSpec((BN, D), lambda i: (i, 0)),
        out_shape=jax.ShapeDtypeStruct((N, D), jnp.float32),
    )(r, d, res, b1)


def kernel(x, edge_index, W0, attn_l0, attn_r0, bias0,
           W1, attn_l1, attn_r1, bias1, W_res1):
    src = edge_index[0]
    dst = edge_index[1]
    al0 = attn_l0.reshape(H0, D)
    ar0 = attn_r0.reshape(H0, D)
    al1 = attn_l1.reshape(1, D)
    ar1 = attn_r1.reshape(1, D)

    ebs, ebd, cnts = _route(src, dst)

    f0, f1, f2, f3, el0, er0 = _tc1(x, W0, al0, ar0)

    rs, ds = [], []
    for h, fe in enumerate((f0, f1, f2, f3)):
        elh = el0[:, h]
        erh = jnp.pad(er0[:, h], (0, NP - N))
        rst, den = _agg(fe, elh, erh, ebs, ebd, cnts)
        rs.append(rst[:N])
        ds.append(den.reshape(NP)[:N][:, None])

    fte1, res1, elr1 = _tc2(rs, ds, bias0.reshape(1, H0 * D),
                            W1, W_res1, al1, ar1)

    el1 = elr1[:, 0]
    er1 = jnp.pad(elr1[:, 1], (0, NP - N))
    rst1, den1 = _agg(fte1, el1, er1, ebs, ebd, cnts)

    return _tc3(rst1[:N], den1.reshape(NP)[:N][:, None],
                res1, bias1.reshape(1, D))


# speedup vs baseline: 24.6379x; 2.3877x over previous
import functools

import jax
import jax.numpy as jnp
from jax import lax
from jax.experimental import pallas as pl
from jax.experimental.pallas import tpu as pltpu
from jax.experimental.pallas import tpu_sc as plsc

N = 10000
E = 320000
D = 256
H0 = 4
NSC = 2
NTILE = 16
NW = NSC * NTILE
NR = 320
NP = NW * NR
EPT = E // NW
CHK = 16000
CAP = 1920
NSLAB = E // CAP + 2
ESLOTS = NSLAB * CAP
G = 48

_sc_mesh = plsc.VectorSubcoreMesh(core_axis_name="c", subcore_axis_name="s")
_sc_params = pltpu.CompilerParams(needs_layout_passes=False)


def _route_body(src_hbm, dst_hbm, ebs_hbm, ebd_hbm, cnt_hbm,
                src_v, dst_v, stgs_v, stgd_v, sem):
    c = lax.axis_index("c")
    s = lax.axis_index("s")
    tid = c * NTILE + s
    lo = tid * NR
    lanes = lax.iota(jnp.int32, 16)

    def chunk(ci, carry):
        off, nslab = carry
        pltpu.sync_copy(src_hbm.at[pl.ds(ci * CHK, CHK)], src_v)
        pltpu.sync_copy(dst_hbm.at[pl.ds(ci * CHK, CHK)], dst_v)

        def step(i, carry):
            off, nslab = carry
            sv = src_v[pl.ds(i * 16, 16)]
            dv = dst_v[pl.ds(i * 16, 16)]
            dl = dv - lo
            m = (dl >= 0) & (dl < NR)
            scan = plsc.cumsum(m.astype(jnp.int32))
            pos = off + scan - 1
            plsc.store_scatter(stgs_v, [pos], sv, mask=m)
            plsc.store_scatter(stgd_v, [pos], dl, mask=m)
            off = off + scan[15]
            full = off >= CAP

            @pl.when(full)
            def _():
                pltpu.sync_copy(stgs_v.at[pl.ds(0, CAP)],
                                ebs_hbm.at[tid, pl.ds(nslab * CAP, CAP)])
                pltpu.sync_copy(stgd_v.at[pl.ds(0, CAP)],
                                ebd_hbm.at[tid, pl.ds(nslab * CAP, CAP)])
                tail_s = stgs_v[pl.ds(CAP, 16)]
                tail_d = stgd_v[pl.ds(CAP, 16)]
                stgs_v[pl.ds(0, 16)] = tail_s
                stgd_v[pl.ds(0, 16)] = tail_d

            off = jnp.where(full, off - CAP, off)
            nslab = jnp.where(full, nslab + 1, nslab)
            return off, nslab

        return lax.fori_loop(0, CHK // 16, step, (off, nslab))

    off, nslab = lax.fori_loop(0, E // CHK, chunk,
                               (jnp.int32(0), jnp.int32(0)))
    pltpu.sync_copy(stgs_v.at[pl.ds(0, CAP)],
                    ebs_hbm.at[tid, pl.ds(nslab * CAP, CAP)])
    pltpu.sync_copy(stgd_v.at[pl.ds(0, CAP)],
                    ebd_hbm.at[tid, pl.ds(nslab * CAP, CAP)])
    cnt = nslab * CAP + off
    for k in range(8):
        stgs_v[pl.ds(k * 16, 16)] = jnp.full((16,), cnt, jnp.int32)
    pltpu.sync_copy(stgs_v.at[pl.ds(0, 128)], cnt_hbm.at[tid])


_route = functools.partial(
    pl.kernel,
    out_type=(
        jax.ShapeDtypeStruct((NW, ESLOTS), jnp.int32),
        jax.ShapeDtypeStruct((NW, ESLOTS), jnp.int32),
        jax.ShapeDtypeStruct((NW, 128), jnp.int32),
    ),
    mesh=_sc_mesh,
    compiler_params=_sc_params,
    scratch_types=[
        pltpu.VMEM((CHK,), jnp.int32),
        pltpu.VMEM((CHK,), jnp.int32),
        pltpu.VMEM((CAP + 16,), jnp.int32),
        pltpu.VMEM((CAP + 16,), jnp.int32),
        pltpu.SemaphoreType.DMA,
    ],
)(_route_body)


def _agg_body(ft_hbm, el_hbm, erp_hbm, ebs_hbm, ebd_hbm, cnt_hbm,
              rst_hbm, den_hbm,
              el_v, er_v, es_v, ed_v, ws_v, rows0_v, rows1_v,
              acc_v, den_v, cnt_v, sem0, sem1):
    c = lax.axis_index("c")
    s = lax.axis_index("s")
    tid = c * NTILE + s
    lanes = lax.iota(jnp.int32, 16)

    pltpu.sync_copy(cnt_hbm.at[tid], cnt_v)
    cnt = cnt_v[pl.ds(0, 16)][0]
    pltpu.sync_copy(el_hbm, el_v)
    pltpu.sync_copy(erp_hbm.at[pl.ds(tid * NR, NR)], er_v)

    zeros16 = jnp.zeros((16,), jnp.float32)

    def zacc(j, _):
        for k in range(D // 16):
            acc_v[j, pl.ds(k * 16, 16)] = zeros16
        return 0
    lax.fori_loop(0, NR, zacc, 0)
    for k in range(NR // 16):
        den_v[pl.ds(k * 16, 16)] = zeros16

    nslabs = (cnt + (CAP - 1)) // CAP

    def accum(rows_ref, base):
        def j16_step(j16, _):
            wvec = ws_v[pl.ds(base + j16 * 16, 16)]
            dvec = ed_v[pl.ds(base + j16 * 16, 16)]
            for jj in range(16):
                wsp = jnp.full((16,), wvec[jj], jnp.float32)
                rowi = jnp.full((16,), dvec[jj], jnp.int32)
                j = j16 * 16 + jj
                for kb in range(D // 128):
                    ks = [kb * 8 + k for k in range(8)]
                    loads = [rows_ref[j, pl.ds(kk * 16, 16)] for kk in ks]
                    prods = [wsp * x for x in loads]
                    for k, kk in enumerate(ks):
                        plsc.addupdate_scatter(acc_v, [rowi, lanes + kk * 16],
                                               prods[k])
            return 0
        lax.fori_loop(0, G // 16, j16_step, 0)

    def slab(sl, _):
        pltpu.sync_copy(ebs_hbm.at[tid, pl.ds(sl * CAP, CAP)], es_v)
        pltpu.sync_copy(ebd_hbm.at[tid, pl.ds(sl * CAP, CAP)], ed_v)

        def wstep(i, _):
            sv = es_v[pl.ds(i * 16, 16)]
            dv = ed_v[pl.ds(i * 16, 16)]
            sv = jnp.minimum(jnp.maximum(sv, 0), N - 1)
            dv = jnp.minimum(jnp.maximum(dv, 0), NR - 1)
            score = (plsc.load_gather(el_v, [sv])
                     + plsc.load_gather(er_v, [dv]))
            score = jnp.where(score >= 0.0, score, 0.2 * score)
            w = jnp.exp(score)
            pos = sl * CAP + i * 16 + lanes
            w = jnp.where(pos < cnt, w, 0.0)
            ws_v[pl.ds(i * 16, 16)] = w
            es_v[pl.ds(i * 16, 16)] = sv
            ed_v[pl.ds(i * 16, 16)] = dv
            plsc.addupdate_scatter(den_v, [dv], w)
            return 0
        lax.fori_loop(0, CAP // 16, wstep, 0)

        rem = cnt - sl * CAP
        ng = jnp.minimum((rem + (G - 1)) // G, CAP // G)

        pltpu.async_copy(ft_hbm.at[es_v.at[pl.ds(0, G)]], rows0_v, sem0)

        def group(g, _):
            nxt = g + 1
            even = (g % 2) == 0

            @pl.when(even)
            def _():
                @pl.when(nxt < ng)
                def _():
                    pltpu.async_copy(ft_hbm.at[es_v.at[pl.ds(nxt * G, G)]],
                                     rows1_v, sem1)
                pltpu.make_async_copy(ft_hbm.at[pl.ds(0, G)],
                                      rows0_v, sem0).wait()
                accum(rows0_v, g * G)

            @pl.when(jnp.logical_not(even))
            def _():
                @pl.when(nxt < ng)
                def _():
                    pltpu.async_copy(ft_hbm.at[es_v.at[pl.ds(nxt * G, G)]],
                                     rows0_v, sem0)
                pltpu.make_async_copy(ft_hbm.at[pl.ds(0, G)],
                                      rows1_v, sem1).wait()
                accum(rows1_v, g * G)
            return 0
        lax.fori_loop(0, ng, group, 0)
        return 0
    lax.fori_loop(0, nslabs, slab, 0)

    pltpu.sync_copy(acc_v, rst_hbm.at[pl.ds(tid * NR, NR)])
    pltpu.sync_copy(den_v, den_hbm.at[tid])


_agg = functools.partial(
    pl.kernel,
    out_type=(
        jax.ShapeDtypeStruct((NP, D), jnp.float32),
        jax.ShapeDtypeStruct((NW, NR), jnp.float32),
    ),
    mesh=_sc_mesh,
    compiler_params=_sc_params,
    scratch_types=[
        pltpu.VMEM((N,), jnp.float32),
        pltpu.VMEM((NR,), jnp.float32),
        pltpu.VMEM((CAP,), jnp.int32),
        pltpu.VMEM((CAP,), jnp.int32),
        pltpu.VMEM((CAP,), jnp.float32),
        pltpu.VMEM((G, D), jnp.float32),
        pltpu.VMEM((G, D), jnp.float32),
        pltpu.VMEM((NR, D), jnp.float32),
        pltpu.VMEM((NR,), jnp.float32),
        pltpu.VMEM((128,), jnp.int32),
        pltpu.SemaphoreType.DMA,
        pltpu.SemaphoreType.DMA,
    ],
)(_agg_body)


BN = 1000


def _tc1_body(x_ref, w_ref, al_ref, ar_ref,
              f0_ref, f1_ref, f2_ref, f3_ref, el_ref, er_ref):
    ft = jnp.dot(x_ref[...], w_ref[...], preferred_element_type=jnp.float32)
    outs = (f0_ref, f1_ref, f2_ref, f3_ref)
    els, ers = [], []
    for h in range(H0):
        fth = ft[:, h * D:(h + 1) * D]
        outs[h][...] = fth
        els.append(jnp.sum(fth * al_ref[h][None, :], axis=1))
        ers.append(jnp.sum(fth * ar_ref[h][None, :], axis=1))
    el_ref[...] = jnp.stack(els, axis=1)
    er_ref[...] = jnp.stack(ers, axis=1)


def _tc1(x, W0, al0, ar0):
    fspec = pl.BlockSpec((BN, D), lambda i: (i, 0))
    espec = pl.BlockSpec((BN, H0), lambda i: (i, 0))
    return pl.pallas_call(
        _tc1_body,
        grid=(N // BN,),
        in_specs=[
            pl.BlockSpec((BN, 128), lambda i: (i, 0)),
            pl.BlockSpec((128, H0 * D), lambda i: (0, 0)),
            pl.BlockSpec((H0, D), lambda i: (0, 0)),
            pl.BlockSpec((H0, D), lambda i: (0, 0)),
        ],
        out_specs=[fspec, fspec, fspec, fspec, espec, espec],
        out_shape=[jax.ShapeDtypeStruct((N, D), jnp.float32)] * H0
        + [jax.ShapeDtypeStruct((N, H0), jnp.float32)] * 2,
    )(x, W0, al0, ar0)


def _elu(v):
    return jnp.where(v > 0.0, v, jnp.exp(jnp.minimum(v, 0.0)) - 1.0)


def _tc2_body(r0_ref, r1_ref, r2_ref, r3_ref,
              d0_ref, d1_ref, d2_ref, d3_ref,
              b0_ref, w1_ref, wr_ref, al_ref, ar_ref,
              ft_ref, res_ref, elr_ref):
    rrefs = (r0_ref, r1_ref, r2_ref, r3_ref)
    drefs = (d0_ref, d1_ref, d2_ref, d3_ref)
    hs = []
    for h in range(H0):
        den = drefs[h][...]
        den = jnp.where(den == 0.0, 1.0, den)
        hf = rrefs[h][...] / den + b0_ref[0, h * D:(h + 1) * D][None, :]
        hs.append(_elu(hf))
    hcat = jnp.concatenate(hs, axis=1)
    ft1 = jnp.dot(hcat, w1_ref[...], preferred_element_type=jnp.float32)
    res = jnp.dot(hcat, wr_ref[...], preferred_element_type=jnp.float32)
    ft_ref[...] = ft1
    res_ref[...] = res
    el1 = jnp.sum(ft1 * al_ref[0][None, :], axis=1)
    er1 = jnp.sum(ft1 * ar_ref[0][None, :], axis=1)
    elr_ref[...] = jnp.concatenate(
        [el1[:, None], er1[:, None], jnp.zeros((BN, 6), jnp.float32)], axis=1)


def _tc2(rs, ds, b0, W1, Wres, al1, ar1):
    rspec = pl.BlockSpec((BN, D), lambda i: (i, 0))
    dspec = pl.BlockSpec((BN, 1), lambda i: (i, 0))
    return pl.pallas_call(
        _tc2_body,
        grid=(N // BN,),
        in_specs=[
            rspec, rspec, rspec, rspec,
            dspec, dspec, dspec, dspec,
            pl.BlockSpec((1, H0 * D), lambda i: (0, 0)),
            pl.BlockSpec((H0 * D, D), lambda i: (0, 0)),
            pl.BlockSpec((H0 * D, D), lambda i: (0, 0)),
            pl.BlockSpec((1, D), lambda i: (0, 0)),
            pl.BlockSpec((1, D), lambda i: (0, 0)),
        ],
        out_specs=[
            pl.BlockSpec((BN, D), lambda i: (i, 0)),
            pl.BlockSpec((BN, D), lambda i: (i, 0)),
            pl.BlockSpec((BN, 8), lambda i: (i, 0)),
        ],
        out_shape=[
            jax.ShapeDtypeStruct((N, D), jnp.float32),
            jax.ShapeDtypeStruct((N, D), jnp.float32),
            jax.ShapeDtypeStruct((N, 8), jnp.float32),
        ],
    )(*rs, *ds, b0, W1, Wres, al1, ar1)


def _tc3_body(r_ref, d_ref, res_ref, b1_ref, out_ref):
    den = d_ref[...]
    den = jnp.where(den == 0.0, 1.0, den)
    o = r_ref[...] / den + res_ref[...] + b1_ref[0][None, :]
    out_ref[...] = _elu(o)


def _tc3(r, d, res, b1):
    return pl.pallas_call(
        _tc3_body,
        grid=(N // BN,),
        in_specs=[
            pl.BlockSpec((BN, D), lambda i: (i, 0)),
            pl.BlockSpec((BN, 1), lambda i: (i, 0)),
            pl.BlockSpec((BN, D), lambda i: (i, 0)),
            pl.BlockSpec((1, D), lambda i: (0, 0)),
        ],
        out_specs=pl.BlockSpec((BN, D), lambda i: (i, 0)),
        out_shape=jax.ShapeDtypeStruct((N, D), jnp.float32),
    )(r, d, res, b1)


def kernel(x, edge_index, W0, attn_l0, attn_r0, bias0,
           W1, attn_l1, attn_r1, bias1, W_res1):
    src = edge_index[0]
    dst = edge_index[1]
    al0 = attn_l0.reshape(H0, D)
    ar0 = attn_r0.reshape(H0, D)
    al1 = attn_l1.reshape(1, D)
    ar1 = attn_r1.reshape(1, D)

    ebs, ebd, cnts = _route(src, dst)

    f0, f1, f2, f3, el0, er0 = _tc1(x, W0, al0, ar0)

    rs, ds = [], []
    for h, fe in enumerate((f0, f1, f2, f3)):
        elh = el0[:, h]
        erh = jnp.pad(er0[:, h], (0, NP - N))
        rst, den = _agg(fe, elh, erh, ebs, ebd, cnts)
        rs.append(rst[:N])
        ds.append(den.reshape(NP)[:N][:, None])

    fte1, res1, elr1 = _tc2(rs, ds, bias0.reshape(1, H0 * D),
                            W1, W_res1, al1, ar1)

    el1 = elr1[:, 0]
    er1 = jnp.pad(elr1[:, 1], (0, NP - N))
    rst1, den1 = _agg(fte1, el1, er1, ebs, ebd, cnts)

    return _tc3(rst1[:N], den1.reshape(NP)[:N][:, None],
                res1, bias1.reshape(1, D))
